# baseline (speedup 1.0000x reference)
import functools

import jax
import jax.numpy as jnp
from jax import lax
from jax.experimental import pallas as pl
from jax.experimental.pallas import tpu as pltpu
from jax.experimental.pallas import tpu_sc as plsc

NC, NS, LANES = 2, 16, 16
NW = NC * NS


def _wid():
    return lax.axis_index("s") * NC + lax.axis_index("c")


def _emb_sum(x_flat, emb, *, n_nodes, L, D):
    CN = 8
    RC = CN * L
    chunks = n_nodes // CN
    iters = (chunks + NW - 1) // NW
    mesh = plsc.VectorSubcoreMesh(core_axis_name="c", subcore_axis_name="s")

    iters += (-iters) % 4

    @functools.partial(
        pl.kernel,
        out_type=jax.ShapeDtypeStruct((n_nodes, D), jnp.float32),
        mesh=mesh,
        scratch_types=[
            pltpu.VMEM((4, RC), jnp.int32),
            pltpu.VMEM((4, RC, D), jnp.float32),
            pltpu.VMEM((2, CN, D), jnp.float32),
            [pltpu.SemaphoreType.DMA] * 2,
            [pltpu.SemaphoreType.DMA] * 2,
            [pltpu.SemaphoreType.DMA] * 2,
        ],
    )
    def k(x_hbm, emb_hbm, out_hbm, idx_v, rows_v, out_v, isem, gsem, osem):
        w = _wid()

        def issue_idx(kk, slot):
            @pl.when(kk < iters)
            def _():
                cid = w + NW * kk

                @pl.when(cid < chunks)
                def _():
                    pltpu.async_copy(
                        x_hbm.at[pl.ds(cid * (CN * L), RC)],
                        idx_v.at[slot], isem[slot % 2])

        def issue_gather(kk, slot):
            cid = w + NW * kk

            @pl.when(cid < chunks)
            def _():
                pltpu.make_async_copy(
                    x_hbm.at[pl.ds(0, RC)], idx_v.at[slot],
                    isem[slot % 2]).wait()
                pltpu.async_copy(emb_hbm.at[idx_v.at[slot]],
                                 rows_v.at[slot], gsem[slot % 2])

        issue_idx(0, 0)
        issue_idx(1, 1)
        issue_idx(2, 2)
        issue_gather(0, 0)
        issue_gather(1, 1)

        @pl.loop(0, iters, step=4)
        def _outer(i):
            for b in range(4):
                kk = i + b
                a = b
                cid = w + NW * kk

                @pl.when(cid < chunks)
                def _():
                    pltpu.make_async_copy(
                        emb_hbm.at[idx_v.at[a]], rows_v.at[a],
                        gsem[a % 2]).wait()

                issue_idx(kk + 3, (b + 3) % 4)
                issue_gather(kk + 2, (b + 2) % 4)

                @pl.when(cid < chunks)
                def _():
                    @pl.when(kk >= 2)
                    def _():
                        pltpu.make_async_copy(
                            out_v.at[a % 2],
                            out_hbm.at[pl.ds(0, CN)], osem[a % 2]).wait()

                    for n in range(CN):
                        for d in range(D // LANES):
                            sl = pl.ds(d * LANES, LANES)
                            acc = rows_v[a, n * L, sl]
                            for r in range(1, L):
                                acc = acc + rows_v[a, n * L + r, sl]
                            out_v[a % 2, n, sl] = acc
                    pltpu.async_copy(out_v.at[a % 2],
                                     out_hbm.at[pl.ds(cid * CN, CN)],
                                     osem[a % 2])

        for kk in (iters - 2, iters - 1):
            @pl.when(w + NW * kk < chunks)
            def _(kk=kk):
                pltpu.make_async_copy(
                    out_v.at[kk % 2], out_hbm.at[pl.ds(0, CN)],
                    osem[kk % 2]).wait()

    return k(x_flat, emb)


def _edge_agg(src, dst, h, *, n_dst, D):
    EC = 128
    n_edges = src.shape[0]
    chunks = n_edges // EC
    iters = (chunks + NW - 1) // NW
    ZR = 40
    nblk = n_dst // ZR
    blk_iters = (nblk + NS - 1) // NS
    mesh = plsc.VectorSubcoreMesh(core_axis_name="c", subcore_axis_name="s")

    iters += iters % 2

    @functools.partial(
        pl.kernel,
        out_type=jax.ShapeDtypeStruct((NC, n_dst, D), jnp.float32),
        mesh=mesh,
        scratch_types=[
            pltpu.VMEM((2, EC), jnp.int32),
            pltpu.VMEM((2, EC), jnp.int32),
            pltpu.VMEM((2, EC, D), jnp.float32),
            pltpu.VMEM((ZR, D), jnp.float32),
            pltpu.VMEM_SHARED((n_dst, D), jnp.float32),
            [pltpu.SemaphoreType.DMA] * 2,
            [pltpu.SemaphoreType.DMA] * 2,
        ],
    )
    def k(src_hbm, dst_hbm, h_hbm, agg_hbm,
          sidx, didx, rows, zb, acc_sh, isem, gsem):
        c = lax.axis_index("c")
        s = lax.axis_index("s")
        w = s * NC + c

        zv = jnp.zeros((LANES,), jnp.float32)

        def fill(i, carry):
            for d in range(D // LANES):
                zb[i, pl.ds(d * LANES, LANES)] = zv
            return carry

        lax.fori_loop(0, ZR, fill, 0)

        def zstep(j, carry):
            bid = s + NS * j

            @pl.when(bid < nblk)
            def _():
                pltpu.sync_copy(zb, acc_sh.at[pl.ds(bid * ZR, ZR)])

            return carry

        lax.fori_loop(0, blk_iters, zstep, 0)
        plsc.subcore_barrier()

        def issue_idx(kk, slot):
            @pl.when((kk < iters) & (w + NW * kk < chunks))
            def _():
                pltpu.async_copy(src_hbm.at[pl.ds((w + NW * kk) * EC, EC)],
                                 sidx.at[slot], isem[slot % 2])
                pltpu.async_copy(dst_hbm.at[pl.ds((w + NW * kk) * EC, EC)],
                                 didx.at[slot], isem[slot % 2])

        def issue_gather(kk, slot):
            @pl.when(w + NW * kk < chunks)
            def _():
                pltpu.make_async_copy(src_hbm.at[pl.ds(0, EC)],
                                      sidx.at[slot], isem[slot % 2]).wait()
                pltpu.make_async_copy(dst_hbm.at[pl.ds(0, EC)],
                                      didx.at[slot], isem[slot % 2]).wait()
                pltpu.async_copy(h_hbm.at[sidx.at[slot]],
                                 rows.at[slot], gsem[slot % 2])

        issue_idx(0, 0)
        issue_idx(1, 1)
        issue_gather(0, 0)

        @pl.loop(0, iters, step=2)
        def _outer(i):
            for b in range(2):
                kk = i + b
                a = b
                na = 1 - b
                cid = w + NW * kk
                issue_gather(kk + 1, na)

                @pl.when(cid < chunks)
                def _():
                    pltpu.make_async_copy(h_hbm.at[sidx.at[a]],
                                          rows.at[a], gsem[a % 2]).wait()
                    pltpu.sync_copy(rows.at[a], acc_sh.at[didx.at[a]],
                                    add=True)

                issue_idx(kk + 2, a)

        plsc.subcore_barrier()

        def wstep(j, carry):
            bid = s + NS * j

            @pl.when(bid < nblk)
            def _():
                pltpu.sync_copy(acc_sh.at[pl.ds(bid * ZR, ZR)],
                                agg_hbm.at[c, pl.ds(bid * ZR, ZR)])

            return carry

        lax.fori_loop(0, blk_iters, wstep, 0)

    return k(src, dst, h)


def _seg_counts(dst1, n1, dst2, n2, *, CW):
    EC = 128
    ZR = 40
    mesh = plsc.VectorSubcoreMesh(core_axis_name="c", subcore_axis_name="s")

    @functools.partial(
        pl.kernel,
        out_type=(
            jax.ShapeDtypeStruct((NC, n1, CW), jnp.float32),
            jax.ShapeDtypeStruct((NC, n2, CW), jnp.float32),
        ),
        mesh=mesh,
        scratch_types=[
            pltpu.VMEM((2, EC), jnp.int32),
            pltpu.VMEM((EC, CW), jnp.float32),
            pltpu.VMEM((ZR, CW), jnp.float32),
            pltpu.VMEM_SHARED((n1, CW), jnp.float32),
            pltpu.VMEM_SHARED((n2, CW), jnp.float32),
            [pltpu.SemaphoreType.DMA] * 2,
        ],
    )
    def k(dst1_hbm, dst2_hbm, cnt1_hbm, cnt2_hbm,
          didx, ones_v, zb, cnt1_sh, cnt2_sh, isem):
        c = lax.axis_index("c")
        s = lax.axis_index("s")
        w = s * NC + c

        zv = jnp.zeros((LANES,), jnp.float32)
        ov = jnp.ones((LANES,), jnp.float32)

        def fill(i, carry):
            for d in range(CW // LANES):
                zb[i, pl.ds(d * LANES, LANES)] = zv
                ones_v[i, pl.ds(d * LANES, LANES)] = ov
            return carry

        lax.fori_loop(0, ZR, fill, 0)

        def fill1(i, carry):
            for d in range(CW // LANES):
                ones_v[i, pl.ds(d * LANES, LANES)] = ov
            return carry

        lax.fori_loop(ZR, EC, fill1, 0)

        for cnt_sh, n_dst in ((cnt1_sh, n1), (cnt2_sh, n2)):
            nblk = n_dst // ZR

            def zstep(j, carry, cnt_sh=cnt_sh, nblk=nblk):
                bid = s + NS * j

                @pl.when(bid < nblk)
                def _():
                    pltpu.sync_copy(zb, cnt_sh.at[pl.ds(bid * ZR, ZR)])

                return carry

            lax.fori_loop(0, (nblk + NS - 1) // NS, zstep, 0)
        plsc.subcore_barrier()

        for cnt_sh, dst_hbm in ((cnt1_sh, dst1_hbm), (cnt2_sh, dst2_hbm)):
            chunks = dst_hbm.shape[0] // EC
            iters = (chunks + NW - 1) // NW
            iters += iters % 2

            def issue_idx(kk, slot, dst_hbm=dst_hbm, chunks=chunks,
                          iters=iters):
                @pl.when((kk < iters) & (w + NW * kk < chunks))
                def _():
                    pltpu.async_copy(
                        dst_hbm.at[pl.ds((w + NW * kk) * EC, EC)],
                        didx.at[slot], isem[slot % 2])

            issue_idx(0, 0)
            issue_idx(1, 1)

            @pl.loop(0, iters, step=2)
            def _outer(i, issue_idx=issue_idx, cnt_sh=cnt_sh,
                       dst_hbm=dst_hbm, chunks=chunks):
                for b in range(2):
                    kk = i + b
                    a = b
                    cid = w + NW * kk

                    @pl.when(cid < chunks)
                    def _():
                        pltpu.make_async_copy(dst_hbm.at[pl.ds(0, EC)],
                                              didx.at[a], isem[a % 2]).wait()
                        pltpu.sync_copy(ones_v, cnt_sh.at[didx.at[a]],
                                        add=True)

                    issue_idx(kk + 2, a)

        plsc.subcore_barrier()

        for cnt_sh, n_dst, cnt_hbm in ((cnt1_sh, n1, cnt1_hbm),
                                       (cnt2_sh, n2, cnt2_hbm)):
            nblk = n_dst // ZR

            def wstep(j, carry, cnt_sh=cnt_sh, cnt_hbm=cnt_hbm, nblk=nblk):
                bid = s + NS * j

                @pl.when(bid < nblk)
                def _():
                    pltpu.sync_copy(cnt_sh.at[pl.ds(bid * ZR, ZR)],
                                    cnt_hbm.at[c, pl.ds(bid * ZR, ZR)])

                return carry

            lax.fori_loop(0, (nblk + NS - 1) // NS, wstep, 0)

    return k(dst1, dst2)


def _ln_relu(h, scale, bias, *, rows_per_block=1000):
    n, d = h.shape

    def body(h_ref, s_ref, b_ref, o_ref):
        x = h_ref[...]
        mu = jnp.mean(x, axis=-1, keepdims=True)
        var = jnp.mean((x - mu) ** 2, axis=-1, keepdims=True)
        y = (x - mu) * lax.rsqrt(var + 1e-5) * s_ref[...] + b_ref[...]
        o_ref[...] = jnp.maximum(y, 0.0)

    grid = n // rows_per_block
    return pl.pallas_call(
        body,
        grid=(grid,),
        in_specs=[
            pl.BlockSpec((rows_per_block, d), lambda i: (i, 0)),
            pl.BlockSpec((1, d), lambda i: (0, 0)),
            pl.BlockSpec((1, d), lambda i: (0, 0)),
        ],
        out_specs=pl.BlockSpec((rows_per_block, d), lambda i: (i, 0)),
        out_shape=jax.ShapeDtypeStruct((n, d), jnp.float32),
    )(h, scale.reshape(1, d), bias.reshape(1, d))


def _sage_dense(aggp, cntp, x_tgt, Wl, bl, Wr, *, relu, rows_per_block=1000):
    n, d = x_tgt.shape
    hh = Wl.shape[1]

    CW = cntp.shape[-1]

    def body(a_ref, c_ref, t_ref, wl_ref, bl_ref, wr_ref, o_ref):
        ssum = a_ref[0] + a_ref[1]
        cnt = jnp.maximum((c_ref[0] + c_ref[1])[:, 0:1], 1.0)
        agg = ssum / cnt
        y = (jnp.dot(agg, wl_ref[...], preferred_element_type=jnp.float32)
             + bl_ref[...]
             + jnp.dot(t_ref[...], wr_ref[...],
                       preferred_element_type=jnp.float32))
        if relu:
            y = jnp.maximum(y, 0.0)
        o_ref[...] = y

    grid = n // rows_per_block
    return pl.pallas_call(
        body,
        grid=(grid,),
        in_specs=[
            pl.BlockSpec((NC, rows_per_block, d), lambda i: (0, i, 0)),
            pl.BlockSpec((NC, rows_per_block, CW), lambda i: (0, i, 0)),
            pl.BlockSpec((rows_per_block, d), lambda i: (i, 0)),
            pl.BlockSpec((d, hh), lambda i: (0, 0)),
            pl.BlockSpec((1, hh), lambda i: (0, 0)),
            pl.BlockSpec((d, hh), lambda i: (0, 0)),
        ],
        out_specs=pl.BlockSpec((rows_per_block, hh), lambda i: (i, 0)),
        out_shape=jax.ShapeDtypeStruct((n, hh), jnp.float32),
    )(aggp, cntp, x_tgt, Wl, bl.reshape(1, hh), Wr)


def kernel(x, edge_index1, edge_index2, emb, ln_scale, ln_bias,
           W1l, b1l, W1r, W2l, b2l, W2r):
    n0, L = x.shape
    V, D = emb.shape
    n1, n2 = 10000, 2000

    x_flat = x.reshape(-1).astype(jnp.int32)
    src1 = edge_index1[0].astype(jnp.int32)
    dst1 = edge_index1[1].astype(jnp.int32)
    src2 = edge_index2[0].astype(jnp.int32)
    dst2 = edge_index2[1].astype(jnp.int32)

    cntp1, cntp2 = _seg_counts(dst1, n1, dst2, n2, CW=128)
    h_raw = _emb_sum(x_flat, emb, n_nodes=n0, L=L, D=D)
    h = _ln_relu(h_raw, ln_scale, ln_bias)
    aggp1 = _edge_agg(src1, dst1, h, n_dst=n1, D=D)
    h1 = _sage_dense(aggp1, cntp1, h[:n1], W1l, b1l, W1r, relu=True)
    aggp2 = _edge_agg(src2, dst2, h1, n_dst=n2, D=D)
    h2 = _sage_dense(aggp2, cntp2, h1[:n2], W2l, b2l, W2r, relu=False)
    return h2


# speedup vs baseline: 1.0359x; 1.0359x over previous
import functools

import jax
import jax.numpy as jnp
from jax import lax
from jax.experimental import pallas as pl
from jax.experimental.pallas import tpu as pltpu
from jax.experimental.pallas import tpu_sc as plsc

NC, NS, LANES = 2, 16, 16
NW = NC * NS


def _wid():
    return lax.axis_index("s") * NC + lax.axis_index("c")


def _emb_sum(x_flat, emb, *, n_nodes, L, D):
    CN = 8
    RC = CN * L
    chunks = n_nodes // CN
    iters = (chunks + NW - 1) // NW
    mesh = plsc.VectorSubcoreMesh(core_axis_name="c", subcore_axis_name="s")

    iters += iters % 2

    @functools.partial(
        pl.kernel,
        out_type=jax.ShapeDtypeStruct((n_nodes, D), jnp.float32),
        mesh=mesh,
        scratch_types=[
            pltpu.VMEM((2, RC), jnp.int32),
            pltpu.VMEM((2, RC, D), jnp.float32),
            pltpu.VMEM((2, CN, D), jnp.float32),
            [pltpu.SemaphoreType.DMA] * 2,
            [pltpu.SemaphoreType.DMA] * 2,
            [pltpu.SemaphoreType.DMA] * 2,
        ],
    )
    def k(x_hbm, emb_hbm, out_hbm, idx_v, rows_v, out_v, isem, gsem, osem):
        w = _wid()

        def issue_idx(kk, slot):
            @pl.when(kk < iters)
            def _():
                cid = w + NW * kk

                @pl.when(cid < chunks)
                def _():
                    pltpu.async_copy(
                        x_hbm.at[pl.ds(cid * (CN * L), RC)],
                        idx_v.at[slot], isem[slot])

        def issue_gather(kk, slot):
            cid = w + NW * kk

            @pl.when(cid < chunks)
            def _():
                pltpu.make_async_copy(
                    x_hbm.at[pl.ds(0, RC)], idx_v.at[slot],
                    isem[slot]).wait()
                pltpu.async_copy(emb_hbm.at[idx_v.at[slot]],
                                 rows_v.at[slot], gsem[slot])

        issue_idx(0, 0)
        issue_idx(1, 1)
        issue_gather(0, 0)

        @pl.loop(0, iters, step=2)
        def _outer(i):
            for b in range(2):
                kk = i + b
                a = b
                na = 1 - b
                cid = w + NW * kk
                issue_gather(kk + 1, na)
                issue_idx(kk + 2, a)

                @pl.when(cid < chunks)
                def _():
                    pltpu.make_async_copy(
                        emb_hbm.at[idx_v.at[a]], rows_v.at[a],
                        gsem[a]).wait()

                    @pl.when(kk >= 2)
                    def _():
                        pltpu.make_async_copy(
                            out_v.at[a],
                            out_hbm.at[pl.ds(0, CN)], osem[a]).wait()

                    for n in range(CN):
                        for d in range(D // LANES):
                            sl = pl.ds(d * LANES, LANES)
                            acc = rows_v[a, n * L, sl]
                            for r in range(1, L):
                                acc = acc + rows_v[a, n * L + r, sl]
                            out_v[a, n, sl] = acc
                    pltpu.async_copy(out_v.at[a],
                                     out_hbm.at[pl.ds(cid * CN, CN)],
                                     osem[a])

        for kk in (iters - 2, iters - 1):
            @pl.when(w + NW * kk < chunks)
            def _(kk=kk):
                pltpu.make_async_copy(
                    out_v.at[kk % 2], out_hbm.at[pl.ds(0, CN)],
                    osem[kk % 2]).wait()

    return k(x_flat, emb)


def _edge_agg(src, dst, h, *, n_dst, D):
    EC = 128
    n_edges = src.shape[0]
    chunks = n_edges // EC
    iters = (chunks + NW - 1) // NW
    ZR = 40
    nblk = n_dst // ZR
    blk_iters = (nblk + NS - 1) // NS
    mesh = plsc.VectorSubcoreMesh(core_axis_name="c", subcore_axis_name="s")

    iters += iters % 2

    @functools.partial(
        pl.kernel,
        out_type=jax.ShapeDtypeStruct((NC, n_dst, D), jnp.float32),
        mesh=mesh,
        scratch_types=[
            pltpu.VMEM((2, EC), jnp.int32),
            pltpu.VMEM((2, EC), jnp.int32),
            pltpu.VMEM((2, EC, D), jnp.float32),
            pltpu.VMEM((ZR, D), jnp.float32),
            pltpu.VMEM_SHARED((n_dst, D), jnp.float32),
            [pltpu.SemaphoreType.DMA] * 2,
            [pltpu.SemaphoreType.DMA] * 2,
        ],
    )
    def k(src_hbm, dst_hbm, h_hbm, agg_hbm,
          sidx, didx, rows, zb, acc_sh, isem, gsem):
        c = lax.axis_index("c")
        s = lax.axis_index("s")
        w = s * NC + c

        zv = jnp.zeros((LANES,), jnp.float32)

        def fill(i, carry):
            for d in range(D // LANES):
                zb[i, pl.ds(d * LANES, LANES)] = zv
            return carry

        lax.fori_loop(0, ZR, fill, 0)

        def zstep(j, carry):
            bid = s + NS * j

            @pl.when(bid < nblk)
            def _():
                pltpu.sync_copy(zb, acc_sh.at[pl.ds(bid * ZR, ZR)])

            return carry

        lax.fori_loop(0, blk_iters, zstep, 0)
        plsc.subcore_barrier()

        def issue_idx(kk, slot):
            @pl.when((kk < iters) & (w + NW * kk < chunks))
            def _():
                pltpu.async_copy(src_hbm.at[pl.ds((w + NW * kk) * EC, EC)],
                                 sidx.at[slot], isem[slot % 2])
                pltpu.async_copy(dst_hbm.at[pl.ds((w + NW * kk) * EC, EC)],
                                 didx.at[slot], isem[slot % 2])

        def issue_gather(kk, slot):
            @pl.when(w + NW * kk < chunks)
            def _():
                pltpu.make_async_copy(src_hbm.at[pl.ds(0, EC)],
                                      sidx.at[slot], isem[slot % 2]).wait()
                pltpu.make_async_copy(dst_hbm.at[pl.ds(0, EC)],
                                      didx.at[slot], isem[slot % 2]).wait()
                pltpu.async_copy(h_hbm.at[sidx.at[slot]],
                                 rows.at[slot], gsem[slot % 2])

        issue_idx(0, 0)
        issue_idx(1, 1)
        issue_gather(0, 0)

        @pl.loop(0, iters, step=2)
        def _outer(i):
            for b in range(2):
                kk = i + b
                a = b
                na = 1 - b
                cid = w + NW * kk
                issue_gather(kk + 1, na)

                @pl.when(cid < chunks)
                def _():
                    pltpu.make_async_copy(h_hbm.at[sidx.at[a]],
                                          rows.at[a], gsem[a % 2]).wait()
                    pltpu.sync_copy(rows.at[a], acc_sh.at[didx.at[a]],
                                    add=True)

                issue_idx(kk + 2, a)

        plsc.subcore_barrier()

        def wstep(j, carry):
            bid = s + NS * j

            @pl.when(bid < nblk)
            def _():
                pltpu.sync_copy(acc_sh.at[pl.ds(bid * ZR, ZR)],
                                agg_hbm.at[c, pl.ds(bid * ZR, ZR)])

            return carry

        lax.fori_loop(0, blk_iters, wstep, 0)

    return k(src, dst, h)


def _seg_counts(dst1, n1, dst2, n2, *, CW):
    EC = 128
    ZR = 40
    mesh = plsc.VectorSubcoreMesh(core_axis_name="c", subcore_axis_name="s")

    @functools.partial(
        pl.kernel,
        out_type=(
            jax.ShapeDtypeStruct((NC, n1, CW), jnp.float32),
            jax.ShapeDtypeStruct((NC, n2, CW), jnp.float32),
        ),
        mesh=mesh,
        scratch_types=[
            pltpu.VMEM((2, EC), jnp.int32),
            pltpu.VMEM((EC, CW), jnp.float32),
            pltpu.VMEM((ZR, CW), jnp.float32),
            pltpu.VMEM_SHARED((n1, CW), jnp.float32),
            pltpu.VMEM_SHARED((n2, CW), jnp.float32),
            [pltpu.SemaphoreType.DMA] * 2,
        ],
    )
    def k(dst1_hbm, dst2_hbm, cnt1_hbm, cnt2_hbm,
          didx, ones_v, zb, cnt1_sh, cnt2_sh, isem):
        c = lax.axis_index("c")
        s = lax.axis_index("s")
        w = s * NC + c

        zv = jnp.zeros((LANES,), jnp.float32)
        ov = jnp.ones((LANES,), jnp.float32)

        def fill(i, carry):
            for d in range(CW // LANES):
                zb[i, pl.ds(d * LANES, LANES)] = zv
                ones_v[i, pl.ds(d * LANES, LANES)] = ov
            return carry

        lax.fori_loop(0, ZR, fill, 0)

        def fill1(i, carry):
            for d in range(CW // LANES):
                ones_v[i, pl.ds(d * LANES, LANES)] = ov
            return carry

        lax.fori_loop(ZR, EC, fill1, 0)

        for cnt_sh, n_dst in ((cnt1_sh, n1), (cnt2_sh, n2)):
            nblk = n_dst // ZR

            def zstep(j, carry, cnt_sh=cnt_sh, nblk=nblk):
                bid = s + NS * j

                @pl.when(bid < nblk)
                def _():
                    pltpu.sync_copy(zb, cnt_sh.at[pl.ds(bid * ZR, ZR)])

                return carry

            lax.fori_loop(0, (nblk + NS - 1) // NS, zstep, 0)
        plsc.subcore_barrier()

        for cnt_sh, dst_hbm in ((cnt1_sh, dst1_hbm), (cnt2_sh, dst2_hbm)):
            chunks = dst_hbm.shape[0] // EC
            iters = (chunks + NW - 1) // NW
            iters += iters % 2

            def issue_idx(kk, slot, dst_hbm=dst_hbm, chunks=chunks,
                          iters=iters):
                @pl.when((kk < iters) & (w + NW * kk < chunks))
                def _():
                    pltpu.async_copy(
                        dst_hbm.at[pl.ds((w + NW * kk) * EC, EC)],
                        didx.at[slot], isem[slot % 2])

            issue_idx(0, 0)
            issue_idx(1, 1)

            @pl.loop(0, iters, step=2)
            def _outer(i, issue_idx=issue_idx, cnt_sh=cnt_sh,
                       dst_hbm=dst_hbm, chunks=chunks):
                for b in range(2):
                    kk = i + b
                    a = b
                    cid = w + NW * kk

                    @pl.when(cid < chunks)
                    def _():
                        pltpu.make_async_copy(dst_hbm.at[pl.ds(0, EC)],
                                              didx.at[a], isem[a % 2]).wait()
                        pltpu.sync_copy(ones_v, cnt_sh.at[didx.at[a]],
                                        add=True)

                    issue_idx(kk + 2, a)

        plsc.subcore_barrier()

        for cnt_sh, n_dst, cnt_hbm in ((cnt1_sh, n1, cnt1_hbm),
                                       (cnt2_sh, n2, cnt2_hbm)):
            nblk = n_dst // ZR

            def wstep(j, carry, cnt_sh=cnt_sh, cnt_hbm=cnt_hbm, nblk=nblk):
                bid = s + NS * j

                @pl.when(bid < nblk)
                def _():
                    pltpu.sync_copy(cnt_sh.at[pl.ds(bid * ZR, ZR)],
                                    cnt_hbm.at[c, pl.ds(bid * ZR, ZR)])

                return carry

            lax.fori_loop(0, (nblk + NS - 1) // NS, wstep, 0)

    return k(dst1, dst2)


def _ln_relu(h, scale, bias, *, rows_per_block=1000):
    n, d = h.shape

    def body(h_ref, s_ref, b_ref, o_ref):
        x = h_ref[...]
        mu = jnp.mean(x, axis=-1, keepdims=True)
        var = jnp.mean((x - mu) ** 2, axis=-1, keepdims=True)
        y = (x - mu) * lax.rsqrt(var + 1e-5) * s_ref[...] + b_ref[...]
        o_ref[...] = jnp.maximum(y, 0.0)

    grid = n // rows_per_block
    return pl.pallas_call(
        body,
        grid=(grid,),
        in_specs=[
            pl.BlockSpec((rows_per_block, d), lambda i: (i, 0)),
            pl.BlockSpec((1, d), lambda i: (0, 0)),
            pl.BlockSpec((1, d), lambda i: (0, 0)),
        ],
        out_specs=pl.BlockSpec((rows_per_block, d), lambda i: (i, 0)),
        out_shape=jax.ShapeDtypeStruct((n, d), jnp.float32),
    )(h, scale.reshape(1, d), bias.reshape(1, d))


def _sage_dense(aggp, cntp, x_tgt, Wl, bl, Wr, *, relu, rows_per_block=1000):
    n, d = x_tgt.shape
    hh = Wl.shape[1]

    CW = cntp.shape[-1]

    def body(a_ref, c_ref, t_ref, wl_ref, bl_ref, wr_ref, o_ref):
        ssum = a_ref[0] + a_ref[1]
        cnt = jnp.maximum((c_ref[0] + c_ref[1])[:, 0:1], 1.0)
        agg = ssum / cnt
        y = (jnp.dot(agg, wl_ref[...], preferred_element_type=jnp.float32)
             + bl_ref[...]
             + jnp.dot(t_ref[...], wr_ref[...],
                       preferred_element_type=jnp.float32))
        if relu:
            y = jnp.maximum(y, 0.0)
        o_ref[...] = y

    grid = n // rows_per_block
    return pl.pallas_call(
        body,
        grid=(grid,),
        in_specs=[
            pl.BlockSpec((NC, rows_per_block, d), lambda i: (0, i, 0)),
            pl.BlockSpec((NC, rows_per_block, CW), lambda i: (0, i, 0)),
            pl.BlockSpec((rows_per_block, d), lambda i: (i, 0)),
            pl.BlockSpec((d, hh), lambda i: (0, 0)),
            pl.BlockSpec((1, hh), lambda i: (0, 0)),
            pl.BlockSpec((d, hh), lambda i: (0, 0)),
        ],
        out_specs=pl.BlockSpec((rows_per_block, hh), lambda i: (i, 0)),
        out_shape=jax.ShapeDtypeStruct((n, hh), jnp.float32),
    )(aggp, cntp, x_tgt, Wl, bl.reshape(1, hh), Wr)


def kernel(x, edge_index1, edge_index2, emb, ln_scale, ln_bias,
           W1l, b1l, W1r, W2l, b2l, W2r):
    n0, L = x.shape
    V, D = emb.shape
    n1, n2 = 10000, 2000

    x_flat = x.reshape(-1).astype(jnp.int32)
    src1 = edge_index1[0].astype(jnp.int32)
    dst1 = edge_index1[1].astype(jnp.int32)
    src2 = edge_index2[0].astype(jnp.int32)
    dst2 = edge_index2[1].astype(jnp.int32)

    cntp1, cntp2 = _seg_counts(dst1, n1, dst2, n2, CW=128)
    h_raw = _emb_sum(x_flat, emb, n_nodes=n0, L=L, D=D)
    h = _ln_relu(h_raw, ln_scale, ln_bias)
    aggp1 = _edge_agg(src1, dst1, h, n_dst=n1, D=D)
    h1 = _sage_dense(aggp1, cntp1, h[:n1], W1l, b1l, W1r, relu=True)
    aggp2 = _edge_agg(src2, dst2, h1, n_dst=n2, D=D)
    h2 = _sage_dense(aggp2, cntp2, h1[:n2], W2l, b2l, W2r, relu=False)
    return h2


# speedup vs baseline: 1.0650x; 1.0281x over previous
import functools

import jax
import jax.numpy as jnp
from jax import lax
from jax.experimental import pallas as pl
from jax.experimental.pallas import tpu as pltpu
from jax.experimental.pallas import tpu_sc as plsc

NC, NS, LANES = 2, 16, 16
NW = NC * NS


def _wid():
    return lax.axis_index("s") * NC + lax.axis_index("c")


def _emb_sum(x_flat, emb, *, n_nodes, L, D):
    CN = 8
    RC = CN * L
    chunks = n_nodes // CN
    iters = (chunks + NW - 1) // NW
    mesh = plsc.VectorSubcoreMesh(core_axis_name="c", subcore_axis_name="s")

    iters += iters % 2

    @functools.partial(
        pl.kernel,
        out_type=jax.ShapeDtypeStruct((n_nodes, D), jnp.float32),
        mesh=mesh,
        scratch_types=[
            pltpu.VMEM((2, RC), jnp.int32),
            pltpu.VMEM((2, RC, D), jnp.float32),
            pltpu.VMEM((2, CN, D), jnp.float32),
            [pltpu.SemaphoreType.DMA] * 2,
            [pltpu.SemaphoreType.DMA] * 2,
            [pltpu.SemaphoreType.DMA] * 2,
        ],
    )
    def k(x_hbm, emb_hbm, out_hbm, idx_v, rows_v, out_v, isem, gsem, osem):
        w = _wid()

        def issue_idx(kk, slot):
            @pl.when(kk < iters)
            def _():
                cid = w + NW * kk

                @pl.when(cid < chunks)
                def _():
                    pltpu.async_copy(
                        x_hbm.at[pl.ds(cid * (CN * L), RC)],
                        idx_v.at[slot], isem[slot])

        def issue_gather(kk, slot):
            cid = w + NW * kk

            @pl.when(cid < chunks)
            def _():
                pltpu.make_async_copy(
                    x_hbm.at[pl.ds(0, RC)], idx_v.at[slot],
                    isem[slot]).wait()
                pltpu.async_copy(emb_hbm.at[idx_v.at[slot]],
                                 rows_v.at[slot], gsem[slot])

        issue_idx(0, 0)
        issue_idx(1, 1)
        issue_gather(0, 0)

        @pl.loop(0, iters, step=2)
        def _outer(i):
            for b in range(2):
                kk = i + b
                a = b
                na = 1 - b
                cid = w + NW * kk
                issue_gather(kk + 1, na)

                @pl.when(cid < chunks)
                def _():
                    pltpu.make_async_copy(
                        emb_hbm.at[idx_v.at[a]], rows_v.at[a],
                        gsem[a]).wait()

                issue_idx(kk + 2, a)

                @pl.when(cid < chunks)
                def _():

                    @pl.when(kk >= 2)
                    def _():
                        pltpu.make_async_copy(
                            out_v.at[a],
                            out_hbm.at[pl.ds(0, CN)], osem[a]).wait()

                    for n in range(CN):
                        for d in range(D // LANES):
                            sl = pl.ds(d * LANES, LANES)
                            acc = rows_v[a, n * L, sl]
                            for r in range(1, L):
                                acc = acc + rows_v[a, n * L + r, sl]
                            out_v[a, n, sl] = acc
                    pltpu.async_copy(out_v.at[a],
                                     out_hbm.at[pl.ds(cid * CN, CN)],
                                     osem[a])

        for kk in (iters - 2, iters - 1):
            @pl.when(w + NW * kk < chunks)
            def _(kk=kk):
                pltpu.make_async_copy(
                    out_v.at[kk % 2], out_hbm.at[pl.ds(0, CN)],
                    osem[kk % 2]).wait()

    return k(x_flat, emb)


def _edge_agg(src, dst, h, *, n_dst, D):
    EC = 128
    n_edges = src.shape[0]
    chunks = n_edges // EC
    iters = (chunks + NW - 1) // NW
    ZR = 40
    nblk = n_dst // ZR
    blk_iters = (nblk + NS - 1) // NS
    mesh = plsc.VectorSubcoreMesh(core_axis_name="c", subcore_axis_name="s")

    iters += iters % 2

    @functools.partial(
        pl.kernel,
        out_type=jax.ShapeDtypeStruct((NC, n_dst, D), jnp.float32),
        mesh=mesh,
        scratch_types=[
            pltpu.VMEM((2, EC), jnp.int32),
            pltpu.VMEM((2, EC), jnp.int32),
            pltpu.VMEM((2, EC, D), jnp.float32),
            pltpu.VMEM((ZR, D), jnp.float32),
            pltpu.VMEM_SHARED((n_dst, D), jnp.float32),
            [pltpu.SemaphoreType.DMA] * 2,
            [pltpu.SemaphoreType.DMA] * 2,
            [pltpu.SemaphoreType.DMA] * 2,
            [pltpu.SemaphoreType.DMA] * 2,
        ],
    )
    def k(src_hbm, dst_hbm, h_hbm, agg_hbm,
          sidx, didx, rows, zb, acc_sh, isem, dsem, gsem, ssem):
        c = lax.axis_index("c")
        s = lax.axis_index("s")
        w = s * NC + c

        zv = jnp.zeros((LANES,), jnp.float32)

        def fill(i, carry):
            for d in range(D // LANES):
                zb[i, pl.ds(d * LANES, LANES)] = zv
            return carry

        lax.fori_loop(0, ZR, fill, 0)

        def zstep(j, carry):
            bid = s + NS * j

            @pl.when(bid < nblk)
            def _():
                pltpu.sync_copy(zb, acc_sh.at[pl.ds(bid * ZR, ZR)])

            return carry

        lax.fori_loop(0, blk_iters, zstep, 0)
        plsc.subcore_barrier()

        def issue_sidx(kk, slot):
            @pl.when((kk < iters) & (w + NW * kk < chunks))
            def _():
                pltpu.async_copy(src_hbm.at[pl.ds((w + NW * kk) * EC, EC)],
                                 sidx.at[slot], isem[slot])

        def issue_didx(kk, slot):
            @pl.when((kk < iters) & (w + NW * kk < chunks))
            def _():
                pltpu.async_copy(dst_hbm.at[pl.ds((w + NW * kk) * EC, EC)],
                                 didx.at[slot], dsem[slot])

        def issue_gather(kk, slot):
            @pl.when(w + NW * kk < chunks)
            def _():
                pltpu.make_async_copy(src_hbm.at[pl.ds(0, EC)],
                                      sidx.at[slot], isem[slot]).wait()
                pltpu.async_copy(h_hbm.at[sidx.at[slot]],
                                 rows.at[slot], gsem[slot])

        issue_sidx(0, 0)
        issue_sidx(1, 1)
        issue_didx(0, 0)
        issue_gather(0, 0)

        @pl.loop(0, iters, step=2)
        def _outer(i):
            for b in range(2):
                kk = i + b
                a = b
                na = 1 - b
                cid = w + NW * kk

                @pl.when((kk >= 1) & (w + NW * (kk - 1) < chunks))
                def _():
                    pltpu.make_async_copy(
                        rows.at[na], acc_sh.at[pl.ds(0, EC)],
                        ssem[na]).wait()

                issue_didx(kk + 1, na)
                issue_gather(kk + 1, na)

                @pl.when(cid < chunks)
                def _():
                    pltpu.make_async_copy(h_hbm.at[sidx.at[a]],
                                          rows.at[a], gsem[a]).wait()
                    pltpu.make_async_copy(dst_hbm.at[pl.ds(0, EC)],
                                          didx.at[a], dsem[a]).wait()
                    pltpu.async_copy(rows.at[a], acc_sh.at[didx.at[a]],
                                     ssem[a], add=True)

                issue_sidx(kk + 2, a)

        @pl.when(w + NW * (iters - 1) < chunks)
        def _():
            pltpu.make_async_copy(
                rows.at[(iters - 1) % 2], acc_sh.at[pl.ds(0, EC)],
                ssem[(iters - 1) % 2]).wait()

        plsc.subcore_barrier()

        def wstep(j, carry):
            bid = s + NS * j

            @pl.when(bid < nblk)
            def _():
                pltpu.sync_copy(acc_sh.at[pl.ds(bid * ZR, ZR)],
                                agg_hbm.at[c, pl.ds(bid * ZR, ZR)])

            return carry

        lax.fori_loop(0, blk_iters, wstep, 0)

    return k(src, dst, h)


def _seg_counts(dst1, n1, dst2, n2, *, CW):
    EC = 128
    ZR = 40
    mesh = plsc.VectorSubcoreMesh(core_axis_name="c", subcore_axis_name="s")

    @functools.partial(
        pl.kernel,
        out_type=(
            jax.ShapeDtypeStruct((NC, n1, CW), jnp.float32),
            jax.ShapeDtypeStruct((NC, n2, CW), jnp.float32),
        ),
        mesh=mesh,
        scratch_types=[
            pltpu.VMEM((2, EC), jnp.int32),
            pltpu.VMEM((EC, CW), jnp.float32),
            pltpu.VMEM((ZR, CW), jnp.float32),
            pltpu.VMEM_SHARED((n1, CW), jnp.float32),
            pltpu.VMEM_SHARED((n2, CW), jnp.float32),
            [pltpu.SemaphoreType.DMA] * 2,
            [pltpu.SemaphoreType.DMA] * 2,
        ],
    )
    def k(dst1_hbm, dst2_hbm, cnt1_hbm, cnt2_hbm,
          didx, ones_v, zb, cnt1_sh, cnt2_sh, isem, ssem):
        c = lax.axis_index("c")
        s = lax.axis_index("s")
        w = s * NC + c

        zv = jnp.zeros((LANES,), jnp.float32)
        ov = jnp.ones((LANES,), jnp.float32)

        def fill(i, carry):
            for d in range(CW // LANES):
                zb[i, pl.ds(d * LANES, LANES)] = zv
                ones_v[i, pl.ds(d * LANES, LANES)] = ov
            return carry

        lax.fori_loop(0, ZR, fill, 0)

        def fill1(i, carry):
            for d in range(CW // LANES):
                ones_v[i, pl.ds(d * LANES, LANES)] = ov
            return carry

        lax.fori_loop(ZR, EC, fill1, 0)

        for cnt_sh, n_dst in ((cnt1_sh, n1), (cnt2_sh, n2)):
            nblk = n_dst // ZR

            def zstep(j, carry, cnt_sh=cnt_sh, nblk=nblk):
                bid = s + NS * j

                @pl.when(bid < nblk)
                def _():
                    pltpu.sync_copy(zb, cnt_sh.at[pl.ds(bid * ZR, ZR)])

                return carry

            lax.fori_loop(0, (nblk + NS - 1) // NS, zstep, 0)
        plsc.subcore_barrier()

        for cnt_sh, dst_hbm in ((cnt1_sh, dst1_hbm), (cnt2_sh, dst2_hbm)):
            chunks = dst_hbm.shape[0] // EC
            iters = (chunks + NW - 1) // NW
            iters += iters % 2

            def issue_idx(kk, slot, dst_hbm=dst_hbm, chunks=chunks,
                          iters=iters):
                @pl.when((kk < iters) & (w + NW * kk < chunks))
                def _():
                    pltpu.async_copy(
                        dst_hbm.at[pl.ds((w + NW * kk) * EC, EC)],
                        didx.at[slot], isem[slot % 2])

            issue_idx(0, 0)

            @pl.loop(0, iters, step=2)
            def _outer(i, issue_idx=issue_idx, cnt_sh=cnt_sh,
                       dst_hbm=dst_hbm, chunks=chunks):
                for b in range(2):
                    kk = i + b
                    a = b
                    na = 1 - b
                    cid = w + NW * kk

                    @pl.when((kk >= 1) & (w + NW * (kk - 1) < chunks))
                    def _():
                        pltpu.make_async_copy(
                            ones_v, cnt_sh.at[pl.ds(0, EC)],
                            ssem[na]).wait()

                    issue_idx(kk + 1, na)

                    @pl.when(cid < chunks)
                    def _():
                        pltpu.make_async_copy(dst_hbm.at[pl.ds(0, EC)],
                                              didx.at[a], isem[a]).wait()
                        pltpu.async_copy(ones_v, cnt_sh.at[didx.at[a]],
                                         ssem[a], add=True)

            @pl.when(w + NW * (iters - 1) < chunks)
            def _(cnt_sh=cnt_sh):
                pltpu.make_async_copy(
                    ones_v, cnt_sh.at[pl.ds(0, EC)],
                    ssem[(iters - 1) % 2]).wait()

        plsc.subcore_barrier()

        for cnt_sh, n_dst, cnt_hbm in ((cnt1_sh, n1, cnt1_hbm),
                                       (cnt2_sh, n2, cnt2_hbm)):
            nblk = n_dst // ZR

            def wstep(j, carry, cnt_sh=cnt_sh, cnt_hbm=cnt_hbm, nblk=nblk):
                bid = s + NS * j

                @pl.when(bid < nblk)
                def _():
                    pltpu.sync_copy(cnt_sh.at[pl.ds(bid * ZR, ZR)],
                                    cnt_hbm.at[c, pl.ds(bid * ZR, ZR)])

                return carry

            lax.fori_loop(0, (nblk + NS - 1) // NS, wstep, 0)

    return k(dst1, dst2)


def _ln_relu(h, scale, bias, *, rows_per_block=1000):
    n, d = h.shape

    def body(h_ref, s_ref, b_ref, o_ref):
        x = h_ref[...]
        mu = jnp.mean(x, axis=-1, keepdims=True)
        var = jnp.mean((x - mu) ** 2, axis=-1, keepdims=True)
        y = (x - mu) * lax.rsqrt(var + 1e-5) * s_ref[...] + b_ref[...]
        o_ref[...] = jnp.maximum(y, 0.0)

    grid = n // rows_per_block
    return pl.pallas_call(
        body,
        grid=(grid,),
        in_specs=[
            pl.BlockSpec((rows_per_block, d), lambda i: (i, 0)),
            pl.BlockSpec((1, d), lambda i: (0, 0)),
            pl.BlockSpec((1, d), lambda i: (0, 0)),
        ],
        out_specs=pl.BlockSpec((rows_per_block, d), lambda i: (i, 0)),
        out_shape=jax.ShapeDtypeStruct((n, d), jnp.float32),
    )(h, scale.reshape(1, d), bias.reshape(1, d))


def _sage_dense(aggp, cntp, x_tgt, Wl, bl, Wr, *, relu, rows_per_block=1000):
    n, d = x_tgt.shape
    hh = Wl.shape[1]

    CW = cntp.shape[-1]

    def body(a_ref, c_ref, t_ref, wl_ref, bl_ref, wr_ref, o_ref):
        ssum = a_ref[0] + a_ref[1]
        cnt = jnp.maximum((c_ref[0] + c_ref[1])[:, 0:1], 1.0)
        agg = ssum / cnt
        y = (jnp.dot(agg, wl_ref[...], preferred_element_type=jnp.float32)
             + bl_ref[...]
             + jnp.dot(t_ref[...], wr_ref[...],
                       preferred_element_type=jnp.float32))
        if relu:
            y = jnp.maximum(y, 0.0)
        o_ref[...] = y

    grid = n // rows_per_block
    return pl.pallas_call(
        body,
        grid=(grid,),
        in_specs=[
            pl.BlockSpec((NC, rows_per_block, d), lambda i: (0, i, 0)),
            pl.BlockSpec((NC, rows_per_block, CW), lambda i: (0, i, 0)),
            pl.BlockSpec((rows_per_block, d), lambda i: (i, 0)),
            pl.BlockSpec((d, hh), lambda i: (0, 0)),
            pl.BlockSpec((1, hh), lambda i: (0, 0)),
            pl.BlockSpec((d, hh), lambda i: (0, 0)),
        ],
        out_specs=pl.BlockSpec((rows_per_block, hh), lambda i: (i, 0)),
        out_shape=jax.ShapeDtypeStruct((n, hh), jnp.float32),
    )(aggp, cntp, x_tgt, Wl, bl.reshape(1, hh), Wr)


def kernel(x, edge_index1, edge_index2, emb, ln_scale, ln_bias,
           W1l, b1l, W1r, W2l, b2l, W2r):
    n0, L = x.shape
    V, D = emb.shape
    n1, n2 = 10000, 2000

    x_flat = x.reshape(-1).astype(jnp.int32)
    src1 = edge_index1[0].astype(jnp.int32)
    dst1 = edge_index1[1].astype(jnp.int32)
    src2 = edge_index2[0].astype(jnp.int32)
    dst2 = edge_index2[1].astype(jnp.int32)

    cntp1, cntp2 = _seg_counts(dst1, n1, dst2, n2, CW=128)
    h_raw = _emb_sum(x_flat, emb, n_nodes=n0, L=L, D=D)
    h = _ln_relu(h_raw, ln_scale, ln_bias)
    aggp1 = _edge_agg(src1, dst1, h, n_dst=n1, D=D)
    h1 = _sage_dense(aggp1, cntp1, h[:n1], W1l, b1l, W1r, relu=True)
    aggp2 = _edge_agg(src2, dst2, h1, n_dst=n2, D=D)
    h2 = _sage_dense(aggp2, cntp2, h1[:n2], W2l, b2l, W2r, relu=False)
    return h2
